# Initial kernel scaffold; baseline (speedup 1.0000x reference)
import jax
import jax.numpy as jnp
from jax.experimental import pallas as pl


def kernel(x, Wq, bq, Wk, bk, Wv, bv, Wp, bp, Wr, br):
    raise NotImplementedError("write your pallas kernel here")


# speedup vs baseline: 5.4708x; 5.4708x over previous
import functools

import jax
import jax.numpy as jnp
from jax.experimental import pallas as pl
from jax.experimental.pallas import tpu as pltpu

_MEMORY_SLOTS = 1024
_MEMORY_TOPK = 32
_RETRIEVAL_WEIGHT = 0.5


def _kv_body(x_ref, wk_ref, bk_ref, wv_ref, bv_ref, k_ref, v_ref, *, pool):
    xb = x_ref[0]
    t, c = xb.shape
    s = t // pool
    if pool == 1:
        slots = xb
    else:
        slots = xb.reshape(s, pool, c).sum(axis=1) * (1.0 / pool)
    nt = (((1,), (1,)), ((), ()))
    k = jax.lax.dot_general(slots, wk_ref[...], nt,
                            preferred_element_type=jnp.float32) + bk_ref[...]
    v = jax.lax.dot_general(slots, wv_ref[...], nt,
                            preferred_element_type=jnp.float32) + bv_ref[...]
    k_ref[0] = k
    v_ref[0] = v


def _avg_floor(a, b):
    return (a & b) + ((a ^ b) >> 1)


def _retrieve_body(x_ref, k_ref, v_ref, wq_ref, bq_ref, wp_ref, bp_ref,
                   wr_ref, br_ref, o_ref, *, topk, inv_sqrt_c):
    xt = x_ref[0]
    nt = (((1,), (1,)), ((), ()))
    q = jax.lax.dot_general(xt, wq_ref[...], nt,
                            preferred_element_type=jnp.float32) + bq_ref[...]
    scores = jax.lax.dot_general(q, k_ref[0], nt,
                                 preferred_element_type=jnp.float32)
    scores = scores * inv_sqrt_c

    bits = jax.lax.bitcast_convert_type(scores, jnp.int32)
    key = jnp.where(bits >= 0, bits, bits ^ jnp.int32(0x7FFFFFFF))

    lo = jnp.min(key, axis=-1, keepdims=True)
    hi = jnp.max(key, axis=-1, keepdims=True)

    def step(_, carry):
        lo, hi = carry
        mid = _avg_floor(lo, hi) + ((lo ^ hi) & 1)
        cnt = jnp.sum((key >= mid).astype(jnp.int32), axis=-1, keepdims=True)
        ge = cnt >= topk
        return jnp.where(ge, mid, lo), jnp.where(ge, hi, mid - 1)

    lo, hi = jax.lax.fori_loop(0, 32, step, (lo, hi))
    m = jnp.max(scores, axis=-1, keepdims=True)
    w = jnp.where(key >= lo, jnp.exp(scores - m), 0.0)
    attn = w * (1.0 / jnp.sum(w, axis=-1, keepdims=True))

    r = jnp.dot(attn, v_ref[0], preferred_element_type=jnp.float32)
    r = jax.lax.dot_general(r, wp_ref[...], nt,
                            preferred_element_type=jnp.float32) + bp_ref[...]

    wd = wr_ref[1:2, :] - wr_ref[0:1, :]
    bd = br_ref[0:1, 1:2] - br_ref[0:1, 0:1]
    gl = jnp.sum(xt * wd, axis=-1, keepdims=True) + bd
    g = jax.nn.sigmoid(gl)

    o_ref[0] = xt + _RETRIEVAL_WEIGHT * g * r


def kernel(x, Wq, bq, Wk, bk, Wv, bv, Wp, bp, Wr, br):
    B, T, C = x.shape
    S = min(T, _MEMORY_SLOTS)
    K = min(_MEMORY_TOPK, S)
    pool = T // S

    bq2 = bq.reshape(1, C)
    bk2 = bk.reshape(1, C)
    bv2 = bv.reshape(1, C)
    bp2 = bp.reshape(1, C)
    br2 = br.reshape(1, 2)

    kv = pl.pallas_call(
        functools.partial(_kv_body, pool=pool),
        grid=(B,),
        in_specs=[
            pl.BlockSpec((1, T, C), lambda b: (b, 0, 0)),
            pl.BlockSpec((C, C), lambda b: (0, 0)),
            pl.BlockSpec((1, C), lambda b: (0, 0)),
            pl.BlockSpec((C, C), lambda b: (0, 0)),
            pl.BlockSpec((1, C), lambda b: (0, 0)),
        ],
        out_specs=[
            pl.BlockSpec((1, S, C), lambda b: (b, 0, 0)),
            pl.BlockSpec((1, S, C), lambda b: (b, 0, 0)),
        ],
        out_shape=[
            jax.ShapeDtypeStruct((B, S, C), jnp.float32),
            jax.ShapeDtypeStruct((B, S, C), jnp.float32),
        ],
        compiler_params=pltpu.CompilerParams(
            dimension_semantics=("parallel",)),
    )
    k, v = kv(x, Wk, bk2, Wv, bv2)

    TB = min(256, T)
    out = pl.pallas_call(
        functools.partial(_retrieve_body, topk=K,
                          inv_sqrt_c=float(1.0 / (C ** 0.5))),
        grid=(B, T // TB),
        in_specs=[
            pl.BlockSpec((1, TB, C), lambda b, t: (b, t, 0)),
            pl.BlockSpec((1, S, C), lambda b, t: (b, 0, 0)),
            pl.BlockSpec((1, S, C), lambda b, t: (b, 0, 0)),
            pl.BlockSpec((C, C), lambda b, t: (0, 0)),
            pl.BlockSpec((1, C), lambda b, t: (0, 0)),
            pl.BlockSpec((C, C), lambda b, t: (0, 0)),
            pl.BlockSpec((1, C), lambda b, t: (0, 0)),
            pl.BlockSpec((2, C), lambda b, t: (0, 0)),
            pl.BlockSpec((1, 2), lambda b, t: (0, 0)),
        ],
        out_specs=pl.BlockSpec((1, TB, C), lambda b, t: (b, t, 0)),
        out_shape=jax.ShapeDtypeStruct((B, T, C), jnp.float32),
        compiler_params=pltpu.CompilerParams(
            dimension_semantics=("parallel", "parallel")),
    )(x, k, v, Wq, bq2, Wp, bp2, Wr, br2)
    return out
